# Initial kernel scaffold; baseline (speedup 1.0000x reference)
import jax
import jax.numpy as jnp
from jax.experimental import pallas as pl


def kernel(x, edge_index, edge_attr, W1, a_src1, a_dst1, b1, g1, be1, W2, a_src2, a_dst2, b2, g2, be2):
    raise NotImplementedError("write your pallas kernel here")


# speedup vs baseline: 24.8824x; 24.8824x over previous
import functools
import jax
import jax.numpy as jnp
from jax import lax
from jax.experimental import pallas as pl
from jax.experimental.pallas import tpu as pltpu
from jax.experimental.pallas import tpu_sc as plsc

N = 100000
E = 1600000
F = 32

NC = 2
NS = 16
HALF = N // NC
ROWS_PER_TILE = 3136
PAD_ROWS = ROWS_PER_TILE * NS
TRASH = HALF
K = 128
CHUNKS = E // K
ITERS = (CHUNKS + NS - 1) // NS


ZCH = ROWS_PER_TILE // 8


def _sc_edge_kernel(h_hbm, asrc_hbm, adst_hbm, src_hbm, dst_hbm,
                    acc_out, den_out,
                    srcv, dstv, sidx, asv, adv, wv, rows, stg2, stg1,
                    acc_sh, den_sh,
                    sem_a, sem_b, sem_c):
    c = lax.axis_index("c")
    s = lax.axis_index("s")
    cbase = c * HALF

    z16 = jnp.zeros((16,), jnp.float32)

    def zloop(j, _):
        stg2[j, pl.ds(0, 16)] = z16
        stg2[j, pl.ds(16, 16)] = z16
        return 0

    lax.fori_loop(0, ZCH, zloop, 0)

    def zloop1(j, _):
        stg1[pl.ds(j * 16, 16)] = z16
        return 0

    lax.fori_loop(0, ROWS_PER_TILE // 16, zloop1, 0)
    r0 = s * ROWS_PER_TILE
    for t in range(8):
        pltpu.sync_copy(stg2, acc_sh.at[pl.ds(r0 + t * ZCH, ZCH)])
    pltpu.sync_copy(stg1, den_sh.at[pl.ds(r0, ROWS_PER_TILE)])
    plsc.subcore_barrier()

    def body(i, _):
        chunk = i * NS + s

        @pl.when(chunk < CHUNKS)
        def _():
            base = chunk * K
            pltpu.sync_copy(src_hbm.at[pl.ds(base, K)], srcv)
            pltpu.sync_copy(dst_hbm.at[pl.ds(base, K)], dstv)
            cp_a = pltpu.async_copy(asrc_hbm.at[srcv], asv, sem_a)
            cp_b = pltpu.async_copy(adst_hbm.at[dstv], adv, sem_b)
            cp_c = pltpu.async_copy(h_hbm.at[srcv], rows, sem_c)
            cp_a.wait()
            cp_b.wait()
            cp_c.wait()
            for j in range(K // 16):
                sl = pl.ds(j * 16, 16)
                e = asv[sl] + adv[sl]
                w16 = jnp.exp(jnp.maximum(e, e * 0.2))
                wv[sl] = w16
                rel = dstv[sl] - cbase
                ok = (rel >= 0) & (rel < HALF)
                sidx[sl] = jnp.where(ok, rel, TRASH)
                for jj in range(16):
                    idx = j * 16 + jj
                    w_s = w16[jj]
                    rows[idx, pl.ds(0, 16)] = rows[idx, pl.ds(0, 16)] * w_s
                    rows[idx, pl.ds(16, 16)] = rows[idx, pl.ds(16, 16)] * w_s
            pltpu.sync_copy(wv, den_sh.at[sidx], add=True)
            pltpu.sync_copy(rows, acc_sh.at[sidx], add=True)
        return 0

    lax.fori_loop(0, ITERS, body, 0)
    plsc.subcore_barrier()

    o0 = c * PAD_ROWS + r0
    for t in range(8):
        pltpu.sync_copy(acc_sh.at[pl.ds(r0 + t * ZCH, ZCH)], stg2)
        pltpu.sync_copy(stg2, acc_out.at[pl.ds(o0 + t * ZCH, ZCH)])
    pltpu.sync_copy(den_sh.at[pl.ds(r0, ROWS_PER_TILE)], stg1)
    pltpu.sync_copy(stg1, den_out.at[pl.ds(o0, ROWS_PER_TILE)])


def _sc_edge_call(h, asrc, adst, src, dst):
    mesh = plsc.VectorSubcoreMesh(core_axis_name="c", subcore_axis_name="s",
                                  num_cores=NC, num_subcores=NS)
    acc, den = pl.kernel(
        _sc_edge_kernel,
        out_type=[
            jax.ShapeDtypeStruct((NC * PAD_ROWS, F), jnp.float32),
            jax.ShapeDtypeStruct((NC * PAD_ROWS,), jnp.float32),
        ],
        mesh=mesh,
        compiler_params=pltpu.CompilerParams(use_tc_tiling_on_sc=False),
        scratch_types=[
            pltpu.VMEM((K,), jnp.int32),
            pltpu.VMEM((K,), jnp.int32),
            pltpu.VMEM((K,), jnp.int32),
            pltpu.VMEM((K,), jnp.float32),
            pltpu.VMEM((K,), jnp.float32),
            pltpu.VMEM((K,), jnp.float32),
            pltpu.VMEM((K, F), jnp.float32),
            pltpu.VMEM((ZCH, F), jnp.float32),
            pltpu.VMEM((ROWS_PER_TILE,), jnp.float32),
            pltpu.VMEM_SHARED((PAD_ROWS, F), jnp.float32),
            pltpu.VMEM_SHARED((PAD_ROWS,), jnp.float32),
            pltpu.SemaphoreType.DMA,
            pltpu.SemaphoreType.DMA,
            pltpu.SemaphoreType.DMA,
        ],
    )(h, asrc, adst, src, dst)
    return acc, den


def _sc_halves(acc, den):
    acc_full = jnp.concatenate([acc[:HALF], acc[PAD_ROWS:PAD_ROWS + HALF]], axis=0)
    den_full = jnp.concatenate([den[:HALF], den[PAD_ROWS:PAD_ROWS + HALF]], axis=0)
    return acc_full, den_full


BLK = 2000
GRID = N // BLK


def _pre1_body(x_ref, w_ref, asr_ref, adr_ref, h_ref, as_ref, ad_ref):
    h = jnp.dot(x_ref[...], w_ref[...], preferred_element_type=jnp.float32)
    h_ref[...] = h
    as_ref[...] = jnp.sum(h * asr_ref[...], axis=1, keepdims=True)
    ad_ref[...] = jnp.sum(h * adr_ref[...], axis=1, keepdims=True)


def _pre1_call(x, W1, a_src, a_dst):
    return pl.pallas_call(
        _pre1_body,
        grid=(GRID,),
        in_specs=[
            pl.BlockSpec((BLK, 10), lambda i: (i, 0)),
            pl.BlockSpec((10, F), lambda i: (0, 0)),
            pl.BlockSpec((1, F), lambda i: (0, 0)),
            pl.BlockSpec((1, F), lambda i: (0, 0)),
        ],
        out_specs=[
            pl.BlockSpec((BLK, F), lambda i: (i, 0)),
            pl.BlockSpec((BLK, 1), lambda i: (i, 0)),
            pl.BlockSpec((BLK, 1), lambda i: (i, 0)),
        ],
        out_shape=[
            jax.ShapeDtypeStruct((N, F), jnp.float32),
            jax.ShapeDtypeStruct((N, 1), jnp.float32),
            jax.ShapeDtypeStruct((N, 1), jnp.float32),
        ],
    )(x, W1, a_src.reshape(1, F), a_dst.reshape(1, F))


def _post_body(acc_ref, den_ref, h_ref, as_ref, ad_ref, b_ref,
               y_ref, sum_ref, ssq_ref):
    i = pl.program_id(0)
    e = as_ref[...] + ad_ref[...]
    w = jnp.exp(jnp.maximum(e, e * 0.2))
    out = (acc_ref[...] + w * h_ref[...]) / (den_ref[...] + w + 1e-16) + b_ref[...]
    y = jnp.maximum(out, 0.0)
    y_ref[...] = y

    @pl.when(i == 0)
    def _():
        sum_ref[...] = jnp.zeros_like(sum_ref)
        ssq_ref[...] = jnp.zeros_like(ssq_ref)

    sum_ref[...] += jnp.sum(y, axis=0, keepdims=True)
    ssq_ref[...] += jnp.sum(y * y, axis=0, keepdims=True)


def _post_call(acc, den, h, as_, ad_, b):
    return pl.pallas_call(
        _post_body,
        grid=(GRID,),
        in_specs=[
            pl.BlockSpec((BLK, F), lambda i: (i, 0)),
            pl.BlockSpec((BLK, 1), lambda i: (i, 0)),
            pl.BlockSpec((BLK, F), lambda i: (i, 0)),
            pl.BlockSpec((BLK, 1), lambda i: (i, 0)),
            pl.BlockSpec((BLK, 1), lambda i: (i, 0)),
            pl.BlockSpec((1, F), lambda i: (0, 0)),
        ],
        out_specs=[
            pl.BlockSpec((BLK, F), lambda i: (i, 0)),
            pl.BlockSpec((1, F), lambda i: (0, 0)),
            pl.BlockSpec((1, F), lambda i: (0, 0)),
        ],
        out_shape=[
            jax.ShapeDtypeStruct((N, F), jnp.float32),
            jax.ShapeDtypeStruct((1, F), jnp.float32),
            jax.ShapeDtypeStruct((1, F), jnp.float32),
        ],
    )(acc, den.reshape(N, 1), h, as_, ad_, b.reshape(1, F))


def _pre2_body(y_ref, w2_ref, s_ref, t_ref, asr_ref, adr_ref,
               h_ref, as_ref, ad_ref):
    ybn = y_ref[...] * s_ref[...] + t_ref[...]
    h = jnp.dot(ybn, w2_ref[...], preferred_element_type=jnp.float32)
    h_ref[...] = h
    as_ref[...] = jnp.sum(h * asr_ref[...], axis=1, keepdims=True)
    ad_ref[...] = jnp.sum(h * adr_ref[...], axis=1, keepdims=True)


def _pre2_call(y, W2, s_row, t_row, a_src, a_dst):
    return pl.pallas_call(
        _pre2_body,
        grid=(GRID,),
        in_specs=[
            pl.BlockSpec((BLK, F), lambda i: (i, 0)),
            pl.BlockSpec((F, F), lambda i: (0, 0)),
            pl.BlockSpec((1, F), lambda i: (0, 0)),
            pl.BlockSpec((1, F), lambda i: (0, 0)),
            pl.BlockSpec((1, F), lambda i: (0, 0)),
            pl.BlockSpec((1, F), lambda i: (0, 0)),
        ],
        out_specs=[
            pl.BlockSpec((BLK, F), lambda i: (i, 0)),
            pl.BlockSpec((BLK, 1), lambda i: (i, 0)),
            pl.BlockSpec((BLK, 1), lambda i: (i, 0)),
        ],
        out_shape=[
            jax.ShapeDtypeStruct((N, F), jnp.float32),
            jax.ShapeDtypeStruct((N, 1), jnp.float32),
            jax.ShapeDtypeStruct((N, 1), jnp.float32),
        ],
    )(y, W2, s_row, t_row, a_src.reshape(1, F), a_dst.reshape(1, F))


def _apply_body(y_ref, s_ref, t_ref, o_ref):
    o_ref[...] = y_ref[...] * s_ref[...] + t_ref[...]


def _apply_call(y, s_row, t_row):
    return pl.pallas_call(
        _apply_body,
        grid=(GRID,),
        in_specs=[
            pl.BlockSpec((BLK, F), lambda i: (i, 0)),
            pl.BlockSpec((1, F), lambda i: (0, 0)),
            pl.BlockSpec((1, F), lambda i: (0, 0)),
        ],
        out_specs=pl.BlockSpec((BLK, F), lambda i: (i, 0)),
        out_shape=jax.ShapeDtypeStruct((N, F), jnp.float32),
    )(y, s_row, t_row)


def _bn_affine(sum_, ssq, g, be, eps=1e-5):
    mu = sum_ / N
    var = ssq / N - mu * mu
    s = g.reshape(1, F) / jnp.sqrt(var + eps)
    t = be.reshape(1, F) - mu * s
    return s, t


@jax.jit
def kernel(x, edge_index, edge_attr, W1, a_src1, a_dst1, b1, g1, be1,
           W2, a_src2, a_dst2, b2, g2, be2):
    src = edge_index[0]
    dst = edge_index[1]

    h1, as1, ad1 = _pre1_call(x, W1, a_src1, a_dst1)
    acc1, den1 = _sc_edge_call(h1, as1.reshape(N), ad1.reshape(N), src, dst)
    acc1, den1 = _sc_halves(acc1, den1)
    y1, sm1, sq1 = _post_call(acc1, den1, h1, as1, ad1, b1)
    s1, t1 = _bn_affine(sm1, sq1, g1, be1)

    h2, as2, ad2 = _pre2_call(y1, W2, s1, t1, a_src2, a_dst2)
    acc2, den2 = _sc_edge_call(h2, as2.reshape(N), ad2.reshape(N), src, dst)
    acc2, den2 = _sc_halves(acc2, den2)
    y2, sm2, sq2 = _post_call(acc2, den2, h2, as2, ad2, b2)
    s2, t2 = _bn_affine(sm2, sq2, g2, be2)

    return _apply_call(y2, s2, t2)


# speedup vs baseline: 38.4705x; 1.5461x over previous
import functools
import jax
import jax.numpy as jnp
from jax import lax
from jax.experimental import pallas as pl
from jax.experimental.pallas import tpu as pltpu
from jax.experimental.pallas import tpu_sc as plsc

N = 100000
E = 1600000
F = 32

NC = 2
NS = 16
HALF = N // NC
ROWS_PER_TILE = 3136
PAD_ROWS = ROWS_PER_TILE * NS
TRASH = HALF
K = 128
CHUNKS = E // K
ITERS = (CHUNKS + NS - 1) // NS


ZCH = ROWS_PER_TILE // 8
EPAD = ITERS * NS * K
LAST = ITERS - 1


def _sc_edge_kernel(h_hbm, asrc_hbm, adst_hbm, ei_hbm,
                    acc_out, den_out,
                    ei0, ei1, sidx0, sidx1, asv0, asv1, adv0, adv1,
                    wv0, wv1, rows0, rows1, stg2, stg1,
                    acc_sh, den_sh,
                    ix0, ix1, ga0, ga1, gb0, gb1, gc0, gc1,
                    sw0, sw1, sr0, sr1):
    c = lax.axis_index("c")
    s = lax.axis_index("s")
    cbase = c * HALF
    eiv = (ei0, ei1)
    sidx = (sidx0, sidx1)
    asv = (asv0, asv1)
    adv = (adv0, adv1)
    wv = (wv0, wv1)
    rows = (rows0, rows1)
    ixs = (ix0, ix1)
    gas = (ga0, ga1)
    gbs = (gb0, gb1)
    gcs = (gc0, gc1)
    sws = (sw0, sw1)
    srs = (sr0, sr1)
    lane = jax.lax.iota(jnp.int32, 16)

    z16 = jnp.zeros((16,), jnp.float32)

    def zloop(j, _):
        stg2[j, pl.ds(0, 16)] = z16
        stg2[j, pl.ds(16, 16)] = z16
        return 0

    lax.fori_loop(0, ZCH, zloop, 0)

    def zloop1(j, _):
        stg1[pl.ds(j * 16, 16)] = z16
        return 0

    lax.fori_loop(0, ROWS_PER_TILE // 16, zloop1, 0)
    r0 = s * ROWS_PER_TILE
    for t in range(8):
        pltpu.sync_copy(stg2, acc_sh.at[pl.ds(r0 + t * ZCH, ZCH)])
    pltpu.sync_copy(stg1, den_sh.at[pl.ds(r0, ROWS_PER_TILE)])
    plsc.subcore_barrier()

    def idx_load(slot, it):
        chunk = it * NS + s
        pltpu.async_copy(ei_hbm.at[:, pl.ds(chunk * K, K)], eiv[slot], ixs[slot])

    def issue_gathers(slot):
        pltpu.async_copy(asrc_hbm.at[eiv[slot].at[0]], asv[slot], gas[slot])
        pltpu.async_copy(adst_hbm.at[eiv[slot].at[1]], adv[slot], gbs[slot])
        pltpu.async_copy(h_hbm.at[eiv[slot].at[0]], rows[slot], gcs[slot])

    def wait_gathers(slot):
        pltpu.make_async_copy(asrc_hbm.at[eiv[slot].at[0]], asv[slot], gas[slot]).wait()
        pltpu.make_async_copy(adst_hbm.at[eiv[slot].at[1]], adv[slot], gbs[slot]).wait()
        pltpu.make_async_copy(h_hbm.at[eiv[slot].at[0]], rows[slot], gcs[slot]).wait()

    def wait_scatters(slot):
        pltpu.make_async_copy(wv[slot], den_sh.at[sidx[slot]], sws[slot]).wait()
        pltpu.make_async_copy(rows[slot], acc_sh.at[sidx[slot]], srs[slot]).wait()

    def compute(slot, it):
        chunk = it * NS + s
        ebase = chunk * K
        for j in range(K // 16):
            sl = pl.ds(j * 16, 16)
            e = asv[slot][sl] + adv[slot][sl]
            w16 = jnp.exp(jnp.maximum(e, e * 0.2))
            wv[slot][sl] = w16
            rel = eiv[slot][1, sl] - cbase
            eids = (ebase + j * 16) + lane
            ok = (rel >= 0) & (rel < HALF) & (eids < E)
            sidx[slot][sl] = jnp.where(ok, rel, TRASH)
            rbuf = rows[slot]
            for jj in range(16):
                idx = j * 16 + jj
                w_s = w16[jj]
                rbuf[idx, pl.ds(0, 16)] = rbuf[idx, pl.ds(0, 16)] * w_s
                rbuf[idx, pl.ds(16, 16)] = rbuf[idx, pl.ds(16, 16)] * w_s

    idx_load(0, 0)
    idx_load(1, 1)
    pltpu.make_async_copy(ei_hbm.at[:, pl.ds(0, K)], eiv[0], ixs[0]).wait()
    issue_gathers(0)

    def body(i2, _):
        for slot in range(2):
            i = i2 * 2 + slot
            nxt = 1 - slot
            wait_gathers(slot)
            compute(slot, i)
            pltpu.async_copy(wv[slot], den_sh.at[sidx[slot]], sws[slot], add=True)
            pltpu.async_copy(rows[slot], acc_sh.at[sidx[slot]], srs[slot], add=True)
            idx_load(slot, jnp.minimum(i + 2, LAST))
            pltpu.make_async_copy(
                ei_hbm.at[:, pl.ds(0, K)], eiv[nxt], ixs[nxt]).wait()
            if slot == 0:
                @pl.when(i2 > 0)
                def _():
                    wait_scatters(nxt)
            else:
                wait_scatters(nxt)
            issue_gathers(nxt)
        return 0

    lax.fori_loop(0, ITERS // 2, body, 0)
    wait_gathers(0)
    wait_scatters(1)
    pltpu.make_async_copy(ei_hbm.at[:, pl.ds(0, K)], eiv[1], ixs[1]).wait()
    plsc.subcore_barrier()

    o0 = c * PAD_ROWS + r0
    for t in range(8):
        pltpu.sync_copy(acc_sh.at[pl.ds(r0 + t * ZCH, ZCH)], stg2)
        pltpu.sync_copy(stg2, acc_out.at[pl.ds(o0 + t * ZCH, ZCH)])
    pltpu.sync_copy(den_sh.at[pl.ds(r0, ROWS_PER_TILE)], stg1)
    pltpu.sync_copy(stg1, den_out.at[pl.ds(o0, ROWS_PER_TILE)])


def _sc_edge_call(h, asrc, adst, ei_pad):
    mesh = plsc.VectorSubcoreMesh(core_axis_name="c", subcore_axis_name="s",
                                  num_cores=NC, num_subcores=NS)
    acc, den = pl.kernel(
        _sc_edge_kernel,
        out_type=[
            jax.ShapeDtypeStruct((NC * PAD_ROWS, F), jnp.float32),
            jax.ShapeDtypeStruct((NC * PAD_ROWS,), jnp.float32),
        ],
        mesh=mesh,
        compiler_params=pltpu.CompilerParams(use_tc_tiling_on_sc=False),
        scratch_types=(
            [pltpu.VMEM((2, K), jnp.int32)] * 2
            + [pltpu.VMEM((K,), jnp.int32)] * 2
            + [pltpu.VMEM((K,), jnp.float32)] * 4
            + [pltpu.VMEM((K,), jnp.float32)] * 2
            + [pltpu.VMEM((K, F), jnp.float32)] * 2
            + [pltpu.VMEM((ZCH, F), jnp.float32),
               pltpu.VMEM((ROWS_PER_TILE,), jnp.float32),
               pltpu.VMEM_SHARED((PAD_ROWS, F), jnp.float32),
               pltpu.VMEM_SHARED((PAD_ROWS,), jnp.float32)]
            + [pltpu.SemaphoreType.DMA] * 12
        ),
    )(h, asrc, adst, ei_pad)
    return acc, den


def _sc_halves(acc, den):
    acc_full = jnp.concatenate([acc[:HALF], acc[PAD_ROWS:PAD_ROWS + HALF]], axis=0)
    den_full = jnp.concatenate([den[:HALF], den[PAD_ROWS:PAD_ROWS + HALF]], axis=0)
    return acc_full, den_full


BLK = 2000
GRID = N // BLK


def _pre1_body(x_ref, w_ref, asr_ref, adr_ref, h_ref, as_ref, ad_ref):
    h = jnp.dot(x_ref[...], w_ref[...], preferred_element_type=jnp.float32)
    h_ref[...] = h
    as_ref[...] = jnp.sum(h * asr_ref[...], axis=1, keepdims=True)
    ad_ref[...] = jnp.sum(h * adr_ref[...], axis=1, keepdims=True)


def _pre1_call(x, W1, a_src, a_dst):
    return pl.pallas_call(
        _pre1_body,
        grid=(GRID,),
        in_specs=[
            pl.BlockSpec((BLK, 10), lambda i: (i, 0)),
            pl.BlockSpec((10, F), lambda i: (0, 0)),
            pl.BlockSpec((1, F), lambda i: (0, 0)),
            pl.BlockSpec((1, F), lambda i: (0, 0)),
        ],
        out_specs=[
            pl.BlockSpec((BLK, F), lambda i: (i, 0)),
            pl.BlockSpec((BLK, 1), lambda i: (i, 0)),
            pl.BlockSpec((BLK, 1), lambda i: (i, 0)),
        ],
        out_shape=[
            jax.ShapeDtypeStruct((N, F), jnp.float32),
            jax.ShapeDtypeStruct((N, 1), jnp.float32),
            jax.ShapeDtypeStruct((N, 1), jnp.float32),
        ],
    )(x, W1, a_src.reshape(1, F), a_dst.reshape(1, F))


def _post_body(acc_ref, den_ref, h_ref, as_ref, ad_ref, b_ref,
               y_ref, sum_ref, ssq_ref):
    i = pl.program_id(0)
    e = as_ref[...] + ad_ref[...]
    w = jnp.exp(jnp.maximum(e, e * 0.2))
    out = (acc_ref[...] + w * h_ref[...]) / (den_ref[...] + w + 1e-16) + b_ref[...]
    y = jnp.maximum(out, 0.0)
    y_ref[...] = y

    @pl.when(i == 0)
    def _():
        sum_ref[...] = jnp.zeros_like(sum_ref)
        ssq_ref[...] = jnp.zeros_like(ssq_ref)

    sum_ref[...] += jnp.sum(y, axis=0, keepdims=True)
    ssq_ref[...] += jnp.sum(y * y, axis=0, keepdims=True)


def _post_call(acc, den, h, as_, ad_, b):
    return pl.pallas_call(
        _post_body,
        grid=(GRID,),
        in_specs=[
            pl.BlockSpec((BLK, F), lambda i: (i, 0)),
            pl.BlockSpec((BLK, 1), lambda i: (i, 0)),
            pl.BlockSpec((BLK, F), lambda i: (i, 0)),
            pl.BlockSpec((BLK, 1), lambda i: (i, 0)),
            pl.BlockSpec((BLK, 1), lambda i: (i, 0)),
            pl.BlockSpec((1, F), lambda i: (0, 0)),
        ],
        out_specs=[
            pl.BlockSpec((BLK, F), lambda i: (i, 0)),
            pl.BlockSpec((1, F), lambda i: (0, 0)),
            pl.BlockSpec((1, F), lambda i: (0, 0)),
        ],
        out_shape=[
            jax.ShapeDtypeStruct((N, F), jnp.float32),
            jax.ShapeDtypeStruct((1, F), jnp.float32),
            jax.ShapeDtypeStruct((1, F), jnp.float32),
        ],
    )(acc, den.reshape(N, 1), h, as_, ad_, b.reshape(1, F))


def _pre2_body(y_ref, w2_ref, s_ref, t_ref, asr_ref, adr_ref,
               h_ref, as_ref, ad_ref):
    ybn = y_ref[...] * s_ref[...] + t_ref[...]
    h = jnp.dot(ybn, w2_ref[...], preferred_element_type=jnp.float32)
    h_ref[...] = h
    as_ref[...] = jnp.sum(h * asr_ref[...], axis=1, keepdims=True)
    ad_ref[...] = jnp.sum(h * adr_ref[...], axis=1, keepdims=True)


def _pre2_call(y, W2, s_row, t_row, a_src, a_dst):
    return pl.pallas_call(
        _pre2_body,
        grid=(GRID,),
        in_specs=[
            pl.BlockSpec((BLK, F), lambda i: (i, 0)),
            pl.BlockSpec((F, F), lambda i: (0, 0)),
            pl.BlockSpec((1, F), lambda i: (0, 0)),
            pl.BlockSpec((1, F), lambda i: (0, 0)),
            pl.BlockSpec((1, F), lambda i: (0, 0)),
            pl.BlockSpec((1, F), lambda i: (0, 0)),
        ],
        out_specs=[
            pl.BlockSpec((BLK, F), lambda i: (i, 0)),
            pl.BlockSpec((BLK, 1), lambda i: (i, 0)),
            pl.BlockSpec((BLK, 1), lambda i: (i, 0)),
        ],
        out_shape=[
            jax.ShapeDtypeStruct((N, F), jnp.float32),
            jax.ShapeDtypeStruct((N, 1), jnp.float32),
            jax.ShapeDtypeStruct((N, 1), jnp.float32),
        ],
    )(y, W2, s_row, t_row, a_src.reshape(1, F), a_dst.reshape(1, F))


def _apply_body(y_ref, s_ref, t_ref, o_ref):
    o_ref[...] = y_ref[...] * s_ref[...] + t_ref[...]


def _apply_call(y, s_row, t_row):
    return pl.pallas_call(
        _apply_body,
        grid=(GRID,),
        in_specs=[
            pl.BlockSpec((BLK, F), lambda i: (i, 0)),
            pl.BlockSpec((1, F), lambda i: (0, 0)),
            pl.BlockSpec((1, F), lambda i: (0, 0)),
        ],
        out_specs=pl.BlockSpec((BLK, F), lambda i: (i, 0)),
        out_shape=jax.ShapeDtypeStruct((N, F), jnp.float32),
    )(y, s_row, t_row)


def _bn_affine(sum_, ssq, g, be, eps=1e-5):
    mu = sum_ / N
    var = ssq / N - mu * mu
    s = g.reshape(1, F) / jnp.sqrt(var + eps)
    t = be.reshape(1, F) - mu * s
    return s, t


@jax.jit
def kernel(x, edge_index, edge_attr, W1, a_src1, a_dst1, b1, g1, be1,
           W2, a_src2, a_dst2, b2, g2, be2):
    ei_pad = jnp.concatenate(
        [edge_index.astype(jnp.int32),
         jnp.zeros((2, EPAD - E), jnp.int32)], axis=1)

    h1, as1, ad1 = _pre1_call(x, W1, a_src1, a_dst1)
    acc1, den1 = _sc_edge_call(h1, as1.reshape(N), ad1.reshape(N), ei_pad)
    acc1, den1 = _sc_halves(acc1, den1)
    y1, sm1, sq1 = _post_call(acc1, den1, h1, as1, ad1, b1)
    s1, t1 = _bn_affine(sm1, sq1, g1, be1)

    h2, as2, ad2 = _pre2_call(y1, W2, s1, t1, a_src2, a_dst2)
    acc2, den2 = _sc_edge_call(h2, as2.reshape(N), ad2.reshape(N), ei_pad)
    acc2, den2 = _sc_halves(acc2, den2)
    y2, sm2, sq2 = _post_call(acc2, den2, h2, as2, ad2, b2)
    s2, t2 = _bn_affine(sm2, sq2, g2, be2)

    return _apply_call(y2, s2, t2)
